# Initial kernel scaffold; baseline (speedup 1.0000x reference)
import jax
import jax.numpy as jnp
from jax.experimental import pallas as pl


def kernel(boxes, scores):
    raise NotImplementedError("write your pallas kernel here")


# speedup vs baseline: 33.5733x; 33.5733x over previous
import functools

import jax
import jax.numpy as jnp
from jax import lax
from jax.experimental import pallas as pl

_N = 5000
_ROWS = 40
_COLS = 128
_PAD = _ROWS * _COLS
_IOU_THRESHOLD = 0.5
_BIG = 1 << 30


def _nms_body(x1_ref, y1_ref, x2_ref, y2_ref, s_ref, keep_ref):
    shape = (_ROWS, _COLS)
    lin = (
        lax.broadcasted_iota(jnp.int32, shape, 0) * _COLS
        + lax.broadcasted_iota(jnp.int32, shape, 1)
    )
    valid = lin < _N

    x1 = x1_ref[...]
    y1 = y1_ref[...]
    x2 = x2_ref[...]
    y2 = y2_ref[...]
    s = s_ref[...]
    area = (x2 - x1) * (y2 - y1)

    neg_inf = jnp.float32(-jnp.inf)
    pos_inf = jnp.float32(jnp.inf)

    s_for_max = jnp.where(valid, s, neg_inf)
    first_max = jnp.max(s_for_max)
    first_min = jnp.min(jnp.where(valid, s, pos_inf))
    sentinel = first_min - 1.0

    idx0 = jnp.min(jnp.where(s_for_max == first_max, lin, _BIG))

    mask0 = valid & (s < first_max)
    copy0 = jnp.where(mask0, s, sentinel)
    keep0 = jnp.zeros(shape, dtype=jnp.float32)
    init_count = jnp.sum(mask0.astype(jnp.int32))

    lane = lax.broadcasted_iota(jnp.int32, (1, _COLS), 1)

    def extract(ref, r, c):
        row = ref[pl.ds(r, 1), :]
        return jnp.max(jnp.where(lane == c, row, neg_inf))

    def cond_fn(state):
        _, _, _, cmax = state
        return (init_count > 0) & (cmax >= first_min)

    def body_fn(state):
        copy, keep, idx, _ = state
        r = idx // _COLS
        c = idx % _COLS
        x1i = extract(x1_ref, r, c)
        y1i = extract(y1_ref, r, c)
        x2i = extract(x2_ref, r, c)
        y2i = extract(y2_ref, r, c)
        ai = (x2i - x1i) * (y2i - y1i)

        xx = jnp.minimum(x2, x2i) - jnp.maximum(x1, x1i)
        yy = jnp.minimum(y2, y2i) - jnp.maximum(y1, y1i)
        inter = jnp.maximum(xx, 0.0) * jnp.maximum(yy, 0.0)
        iou = inter / ((area + ai) - inter)

        onehot = lin == idx
        mask = (copy >= first_min) & (iou <= _IOU_THRESHOLD) & (~onehot)
        keep = jnp.where(onehot, 1.0, keep)
        copy = jnp.where(mask, copy, sentinel)
        cmax = jnp.max(copy)
        idx = jnp.min(jnp.where(copy == cmax, lin, _BIG))
        return copy, keep, idx, cmax

    state = (copy0, keep0, idx0, first_max)
    _, keep, _, _ = lax.while_loop(cond_fn, body_fn, state)
    keep_ref[...] = keep


@functools.partial(jax.jit)
def kernel(boxes, scores):
    pad = _PAD - _N

    def prep(v):
        return jnp.pad(v, (0, pad)).reshape(_ROWS, _COLS)

    x1 = prep(boxes[:, 0])
    y1 = prep(boxes[:, 1])
    x2 = prep(boxes[:, 2])
    y2 = prep(boxes[:, 3])
    s = prep(scores)

    keep = pl.pallas_call(
        _nms_body,
        out_shape=jax.ShapeDtypeStruct((_ROWS, _COLS), jnp.float32),
    )(x1, y1, x2, y2, s)

    m = keep.reshape(_PAD)[:_N]
    return jnp.concatenate([boxes, scores[:, None]], axis=1) * m[:, None]


# speedup vs baseline: 45.3537x; 1.3509x over previous
import functools

import jax
import jax.numpy as jnp
from jax import lax
from jax.experimental import pallas as pl
from jax.experimental.pallas import tpu as pltpu

_N = 5000
_ROWS = 40
_COLS = 128
_PAD = _ROWS * _COLS
_IOU_THRESHOLD = 0.5
_BIG = 1 << 30


def _nms_body(x1_ref, y1_ref, x2_ref, y2_ref, s_ref,
              x1s_ref, y1s_ref, x2s_ref, y2s_ref, keep_ref):
    shape = (_ROWS, _COLS)
    lin = (
        lax.broadcasted_iota(jnp.int32, shape, 0) * _COLS
        + lax.broadcasted_iota(jnp.int32, shape, 1)
    )
    valid = lin < _N

    x1 = x1_ref[...]
    y1 = y1_ref[...]
    x2 = x2_ref[...]
    y2 = y2_ref[...]
    s = s_ref[...]
    area = (x2 - x1) * (y2 - y1)

    neg_inf = jnp.float32(-jnp.inf)
    pos_inf = jnp.float32(jnp.inf)

    s_for_max = jnp.where(valid, s, neg_inf)
    first_max = jnp.max(s_for_max)
    first_min = jnp.min(jnp.where(valid, s, pos_inf))
    sentinel = first_min - 1.0

    idx0 = jnp.min(jnp.where(s_for_max == first_max, lin, _BIG))

    mask0 = valid & (s < first_max)
    copy0 = jnp.where(mask0, s, sentinel)
    keep0 = jnp.zeros(shape, dtype=jnp.float32)
    init_count = jnp.sum(mask0.astype(jnp.int32))

    def cond_fn(state):
        _, _, _, cmax = state
        return (init_count > 0) & (cmax >= first_min)

    def body_fn(state):
        copy, keep, idx, _ = state
        x1i = x1s_ref[idx]
        y1i = y1s_ref[idx]
        x2i = x2s_ref[idx]
        y2i = y2s_ref[idx]
        ai = (x2i - x1i) * (y2i - y1i)

        xx = jnp.minimum(x2, x2i) - jnp.maximum(x1, x1i)
        yy = jnp.minimum(y2, y2i) - jnp.maximum(y1, y1i)
        inter = jnp.maximum(xx, 0.0) * jnp.maximum(yy, 0.0)
        iou = inter / ((area + ai) - inter)

        onehot = lin == idx
        mask = (copy >= first_min) & (iou <= _IOU_THRESHOLD) & (~onehot)
        keep = jnp.where(onehot, 1.0, keep)
        copy = jnp.where(mask, copy, sentinel)
        cmax = jnp.max(copy)
        idx = jnp.min(jnp.where(copy == cmax, lin, _BIG))
        return copy, keep, idx, cmax

    state = (copy0, keep0, idx0, first_max)
    _, keep, _, _ = lax.while_loop(cond_fn, body_fn, state)
    keep_ref[...] = keep


@functools.partial(jax.jit)
def kernel(boxes, scores):
    pad = _PAD - _N

    def prep(v):
        return jnp.pad(v, (0, pad)).reshape(_ROWS, _COLS)

    x1 = prep(boxes[:, 0])
    y1 = prep(boxes[:, 1])
    x2 = prep(boxes[:, 2])
    y2 = prep(boxes[:, 3])
    s = prep(scores)

    x1f = jnp.pad(boxes[:, 0], (0, pad))
    y1f = jnp.pad(boxes[:, 1], (0, pad))
    x2f = jnp.pad(boxes[:, 2], (0, pad))
    y2f = jnp.pad(boxes[:, 3], (0, pad))

    vspec = pl.BlockSpec(memory_space=pltpu.VMEM)
    sspec = pl.BlockSpec(memory_space=pltpu.SMEM)
    keep = pl.pallas_call(
        _nms_body,
        out_shape=jax.ShapeDtypeStruct((_ROWS, _COLS), jnp.float32),
        in_specs=[vspec, vspec, vspec, vspec, vspec,
                  sspec, sspec, sspec, sspec],
    )(x1, y1, x2, y2, s, x1f, y1f, x2f, y2f)

    m = keep.reshape(_PAD)[:_N]
    return jnp.concatenate([boxes, scores[:, None]], axis=1) * m[:, None]
